# Initial kernel scaffold; baseline (speedup 1.0000x reference)
import jax
import jax.numpy as jnp
from jax.experimental import pallas as pl


def kernel(inputs, table):
    raise NotImplementedError("write your pallas kernel here")


# speedup vs baseline: 1.0689x; 1.0689x over previous
import functools

import jax
import jax.numpy as jnp
from jax import lax
from jax.experimental import pallas as pl
from jax.experimental.pallas import tpu as pltpu
from jax.experimental.pallas import tpu_sc as plsc

NC = 2
NS = 16
NW = NC * NS
L = 16
G = 128


@functools.lru_cache(maxsize=None)
def _build(B, V, D):
    rows_per_w = B // NW
    idx_rows_per_w = rows_per_w // G

    mesh = plsc.VectorSubcoreMesh(
        core_axis_name="c", subcore_axis_name="s",
        num_cores=NC, num_subcores=NS)

    @functools.partial(
        pl.kernel,
        out_type=jax.ShapeDtypeStruct((B, D), jnp.float32),
        mesh=mesh,
        scratch_types=[
            pltpu.VMEM((idx_rows_per_w, G), jnp.int32),
            pltpu.VMEM((G, D), jnp.float32),
            pltpu.SemaphoreType.DMA,
        ],
        compiler_params=pltpu.CompilerParams(use_tc_tiling_on_sc=False),
    )
    def emb(idx_hbm, table_hbm, out_hbm, idx_v, buf, gsem):
        wid = lax.axis_index("s") * NC + lax.axis_index("c")
        obase = wid * rows_per_w

        pltpu.sync_copy(idx_hbm.at[pl.ds(wid * idx_rows_per_w, idx_rows_per_w)],
                        idx_v)

        def body(c, carry):
            pltpu.async_copy(table_hbm.at[idx_v.at[c]], buf, gsem).wait()
            pltpu.sync_copy(buf, out_hbm.at[pl.ds(obase + c * G, G)])
            return carry

        lax.fori_loop(0, idx_rows_per_w, body, 0)

    return emb


def kernel(inputs, table):
    V, D = table.shape
    B = inputs.shape[0] * inputs.shape[1]
    idx = inputs.reshape(B // G, G).astype(jnp.int32)
    out = _build(B, V, D)(idx, table)
    return out.reshape(inputs.shape + (D,))


# speedup vs baseline: 1.8304x; 1.7125x over previous
import functools

import jax
import jax.numpy as jnp
from jax import lax
from jax.experimental import pallas as pl
from jax.experimental.pallas import tpu as pltpu
from jax.experimental.pallas import tpu_sc as plsc

NC = 2
NS = 16
NW = NC * NS
L = 16
R = 8
SP = 56


@functools.lru_cache(maxsize=None)
def _build(N, S, V, D):
    rows_per_w = N // NW
    idx_per_w = rows_per_w * SP
    n_chunks = rows_per_w // R
    n2 = n_chunks // 2
    offs = (0, 16, 32, 40)
    assert rows_per_w % R == 0 and n_chunks % 2 == 0

    mesh = plsc.VectorSubcoreMesh(
        core_axis_name="c", subcore_axis_name="s",
        num_cores=NC, num_subcores=NS)

    @functools.partial(
        pl.kernel,
        out_type=jax.ShapeDtypeStruct((N, S, D), jnp.float32),
        mesh=mesh,
        scratch_types=[
            pltpu.VMEM((idx_per_w,), jnp.int32),
            pltpu.VMEM((R, S, D), jnp.float32),
            pltpu.VMEM((R, S, D), jnp.float32),
            pltpu.VMEM((L,), jnp.int32),
            pltpu.SemaphoreType.DMA,
            pltpu.SemaphoreType.DMA,
            pltpu.SemaphoreType.DMA,
            pltpu.SemaphoreType.DMA,
        ],
        compiler_params=pltpu.CompilerParams(
            use_tc_tiling_on_sc=False, needs_layout_passes=False),
    )
    def emb(idx_hbm, table_hbm, out_hbm, idx_v, buf0, buf1, zscr,
            gsem0, gsem1, osem0, osem1):
        wid = lax.axis_index("s") * NC + lax.axis_index("c")
        rbase = wid * rows_per_w

        pltpu.sync_copy(idx_hbm.at[pl.ds(wid * idx_per_w, idx_per_w)], idx_v)

        def fire_gathers(c, buf, sem):
            for r in range(R):
                pltpu.async_copy(
                    table_hbm.at[idx_v.at[pl.ds((c * R + r) * SP, S)]],
                    buf.at[r], sem)

        def wait_gathers(c, buf, sem):
            for r in range(R):
                pltpu.make_async_copy(
                    table_hbm.at[idx_v.at[pl.ds((c * R + r) * SP, S)]],
                    buf.at[r], sem).wait()

        def fire_scatter(c, buf, sem):
            pltpu.async_copy(buf, out_hbm.at[pl.ds(rbase + c * R, R)], sem)

        def wait_scatter(c, buf, sem):
            pltpu.make_async_copy(
                buf, out_hbm.at[pl.ds(rbase + c * R, R)], sem).wait()

        def fix_zeros(c, buf):
            zany = jnp.zeros((L,), jnp.int32)
            for r in range(R):
                for o in offs:
                    v = idx_v[pl.ds((c * R + r) * SP + o, L)]
                    zany = zany | (v == 0).astype(jnp.int32)
            nz = zany[0]
            for k in range(1, L):
                nz = nz + zany[k]

            @pl.when(nz > 0)
            def _():
                zero = jnp.zeros((L,), jnp.float32)
                for r in range(R):
                    for o in offs:
                        v = idx_v[pl.ds((c * R + r) * SP + o, L)]
                        msk = v == 0
                        rows = jnp.zeros((L,), jnp.int32) + r
                        cols = o + lax.iota(jnp.int32, L)

                        def fbody(f, fvec):
                            plsc.store_scatter(buf, [rows, cols, fvec],
                                               zero, mask=msk)
                            return fvec + 1
                        lax.fori_loop(0, D, fbody,
                                      jnp.zeros((L,), jnp.int32))

        fire_gathers(0, buf0, gsem0)

        def body(i, carry):
            a = 2 * i
            b = a + 1
            wait_gathers(a, buf0, gsem0)
            fix_zeros(a, buf0)

            @pl.when(i > 0)
            def _():
                wait_scatter(b - 2, buf1, osem1)
            fire_gathers(b, buf1, gsem1)
            fire_scatter(a, buf0, osem0)

            wait_gathers(b, buf1, gsem1)
            fix_zeros(b, buf1)
            wait_scatter(a, buf0, osem0)

            @pl.when(i < n2 - 1)
            def _():
                fire_gathers(a + 2, buf0, gsem0)
            fire_scatter(b, buf1, osem1)
            return carry

        lax.fori_loop(0, n2, body, 0)
        wait_scatter(2 * n2 - 1, buf1, osem1)

    return emb


def kernel(inputs, table):
    V, D = table.shape
    N, S = inputs.shape
    idx = jnp.pad(inputs.astype(jnp.int32), ((0, 0), (0, SP - S)),
                  constant_values=1).reshape(-1)
    return _build(N, S, V, D)(idx, table)


# speedup vs baseline: 1.9764x; 1.0798x over previous
import functools

import jax
import jax.numpy as jnp
from jax import lax
from jax.experimental import pallas as pl
from jax.experimental.pallas import tpu as pltpu
from jax.experimental.pallas import tpu_sc as plsc

NC = 2
NS = 16
NW = NC * NS
L = 16
G = 128


@functools.lru_cache(maxsize=None)
def _build(N, S, V, D):
    b_per_w = N // NW
    ng = b_per_w // G
    n2 = S // 2
    assert b_per_w % G == 0 and S % 2 == 0

    mesh = plsc.VectorSubcoreMesh(
        core_axis_name="c", subcore_axis_name="s",
        num_cores=NC, num_subcores=NS)

    @functools.partial(
        pl.kernel,
        out_type=jax.ShapeDtypeStruct((S, N, D), jnp.float32),
        mesh=mesh,
        scratch_types=[
            pltpu.VMEM((S, b_per_w), jnp.int32),
            pltpu.VMEM((b_per_w, D), jnp.float32),
            pltpu.VMEM((b_per_w, D), jnp.float32),
            pltpu.SemaphoreType.DMA,
            pltpu.SemaphoreType.DMA,
            pltpu.SemaphoreType.DMA,
            pltpu.SemaphoreType.DMA,
        ],
        compiler_params=pltpu.CompilerParams(
            use_tc_tiling_on_sc=False, needs_layout_passes=False),
    )
    def emb(idx_hbm, table_hbm, out_hbm, idx_v, buf0, buf1,
            gsem0, gsem1, osem0, osem1):
        wid = lax.axis_index("s") * NC + lax.axis_index("c")
        b0 = wid * b_per_w

        pltpu.sync_copy(idx_hbm.at[pl.ds(0, S), pl.ds(b0, b_per_w)], idx_v)

        def fire_gathers(a, buf, sem):
            for k in range(ng):
                pltpu.async_copy(
                    table_hbm.at[idx_v.at[a, pl.ds(k * G, G)]],
                    buf.at[pl.ds(k * G, G)], sem)

        def wait_gathers(a, buf, sem):
            for k in range(ng):
                pltpu.make_async_copy(
                    table_hbm.at[idx_v.at[a, pl.ds(k * G, G)]],
                    buf.at[pl.ds(k * G, G)], sem).wait()

        def fire_scatter(a, buf, sem):
            pltpu.async_copy(buf, out_hbm.at[a, pl.ds(b0, b_per_w)], sem)

        def wait_scatter(a, buf, sem):
            pltpu.make_async_copy(
                buf, out_hbm.at[a, pl.ds(b0, b_per_w)], sem).wait()

        def fix_zeros(a, buf):
            zany = jnp.zeros((L,), jnp.int32)
            for j in range(b_per_w // L):
                v = idx_v[a, pl.ds(j * L, L)]
                zany = zany | (v == 0).astype(jnp.int32)
            nz = zany[0]
            for k in range(1, L):
                nz = nz + zany[k]

            @pl.when(nz > 0)
            def _():
                zero = jnp.zeros((L,), jnp.float32)
                for j in range(b_per_w // L):
                    v = idx_v[a, pl.ds(j * L, L)]
                    msk = v == 0
                    rows = j * L + lax.iota(jnp.int32, L)

                    def fbody(f, fvec):
                        plsc.store_scatter(buf, [rows, fvec], zero, mask=msk)
                        return fvec + 1
                    lax.fori_loop(0, D, fbody, jnp.zeros((L,), jnp.int32))

        fire_gathers(0, buf0, gsem0)

        def body(i, carry):
            a = 2 * i
            b = a + 1
            wait_gathers(a, buf0, gsem0)
            fix_zeros(a, buf0)

            @pl.when(i > 0)
            def _():
                wait_scatter(b - 2, buf1, osem1)
            fire_gathers(b, buf1, gsem1)
            fire_scatter(a, buf0, osem0)

            wait_gathers(b, buf1, gsem1)
            fix_zeros(b, buf1)
            wait_scatter(a, buf0, osem0)

            @pl.when(i < n2 - 1)
            def _():
                fire_gathers(a + 2, buf0, gsem0)
            fire_scatter(b, buf1, osem1)
            return carry

        lax.fori_loop(0, n2, body, 0)
        wait_scatter(2 * n2 - 1, buf1, osem1)

    return emb


def kernel(inputs, table):
    V, D = table.shape
    N, S = inputs.shape
    out_t = _build(N, S, V, D)(inputs.T.astype(jnp.int32), table)
    return jnp.transpose(out_t, (1, 0, 2))
